# Initial kernel scaffold; baseline (speedup 1.0000x reference)
import jax
import jax.numpy as jnp
from jax.experimental import pallas as pl


def kernel(x, W, We):
    raise NotImplementedError("write your pallas kernel here")


# speedup vs baseline: 10.2174x; 10.2174x over previous
import functools

import jax
import jax.numpy as jnp
from jax import lax
from jax.experimental import pallas as pl
from jax.experimental.pallas import tpu as pltpu
from jax.experimental.pallas import tpu_sc as plsc

VOCAB = 1000000
EMB = 128
FACT = 32
N_TOKENS = 16384 * 50

NC = 2
NS = 16
NW = NC * NS
BPW = N_TOKENS // NW
CH = 128
NCH = BPW // CH
K_INFLIGHT = 8
SLAB = CH * K_INFLIGHT


def _gather_sc(x2d, W):
    mesh = plsc.VectorSubcoreMesh(core_axis_name="c", subcore_axis_name="s")

    @functools.partial(
        pl.kernel,
        mesh=mesh,
        compiler_params=pltpu.CompilerParams(use_tc_tiling_on_sc=False),
        out_type=jax.ShapeDtypeStruct((N_TOKENS, FACT), jnp.float32),
        scratch_types=[
            pltpu.VMEM((NCH, CH), jnp.int32),
            pltpu.VMEM((SLAB, FACT), jnp.float32),
            pltpu.SemaphoreType.DMA,
        ],
    )
    def k(x_hbm, w_hbm, out_hbm, idx_v, rows_v, sem):
        wid = lax.axis_index("s") * NC + lax.axis_index("c")
        row0 = wid * BPW
        pltpu.sync_copy(x_hbm.at[pl.ds(wid * NCH, NCH), :], idx_v)
        for s in range(NCH // K_INFLIGHT):
            copies = []
            for j in range(K_INFLIGHT):
                c = s * K_INFLIGHT + j
                copies.append(
                    pltpu.async_copy(
                        w_hbm.at[idx_v.at[c]],
                        rows_v.at[pl.ds(j * CH, CH)],
                        sem,
                    )
                )
            for cp in copies:
                cp.wait()
            pltpu.sync_copy(
                rows_v, out_hbm.at[pl.ds(row0 + s * SLAB, SLAB), :]
            )

    return k(x2d, W)


def _project_tc(g, WeT):
    BLK = 2048

    def mm(g_ref, wt_ref, o_ref):
        o_ref[:, :] = jnp.dot(
            g_ref[:, :], wt_ref[:, :], preferred_element_type=jnp.float32
        )

    return pl.pallas_call(
        mm,
        grid=(N_TOKENS // BLK,),
        in_specs=[
            pl.BlockSpec((BLK, FACT), lambda i: (i, 0)),
            pl.BlockSpec((FACT, EMB), lambda i: (0, 0)),
        ],
        out_specs=pl.BlockSpec((BLK, EMB), lambda i: (i, 0)),
        out_shape=jax.ShapeDtypeStruct((N_TOKENS, EMB), jnp.float32),
    )(g, WeT)


def kernel(x, W, We):
    B, L = x.shape
    x2d = x.reshape(NW * NCH, CH)
    g = _gather_sc(x2d, W)
    out = _project_tc(g, We.T)
    return out.reshape(B, L, EMB)


# speedup vs baseline: 13.3035x; 1.3020x over previous
import functools

import jax
import jax.numpy as jnp
from jax import lax
from jax.experimental import pallas as pl
from jax.experimental.pallas import tpu as pltpu
from jax.experimental.pallas import tpu_sc as plsc

VOCAB = 1000000
EMB = 128
FACT = 32
N_TOKENS = 16384 * 50

NC = 2
NS = 16
NW = NC * NS
BPW = N_TOKENS // NW
CH = 128
NCH = BPW // CH
K_INFLIGHT = 8
SLAB = CH * K_INFLIGHT


def _gather_sc(x2d, W):
    mesh = plsc.VectorSubcoreMesh(core_axis_name="c", subcore_axis_name="s")

    @functools.partial(
        pl.kernel,
        mesh=mesh,
        compiler_params=pltpu.CompilerParams(use_tc_tiling_on_sc=False),
        out_type=jax.ShapeDtypeStruct((N_TOKENS, FACT), jnp.float32),
        scratch_types=[
            pltpu.VMEM((NCH, CH), jnp.int32),
            pltpu.VMEM((SLAB, FACT), jnp.float32),
            pltpu.SemaphoreType.DMA,
        ],
    )
    def k(x_hbm, w_hbm, out_hbm, idx_v, rows_v, sem):
        wid = lax.axis_index("s") * NC + lax.axis_index("c")
        row0 = wid * BPW
        pltpu.sync_copy(x_hbm.at[pl.ds(wid * NCH, NCH), :], idx_v)
        for s in range(NCH // K_INFLIGHT):
            copies = []
            for j in range(K_INFLIGHT):
                c = s * K_INFLIGHT + j
                copies.append(
                    pltpu.async_copy(
                        w_hbm.at[idx_v.at[c]],
                        rows_v.at[pl.ds(j * CH, CH)],
                        sem,
                    )
                )
            for cp in copies:
                cp.wait()
            pltpu.sync_copy(
                rows_v, out_hbm.at[pl.ds(row0 + s * SLAB, SLAB), :]
            )

    return k(x2d, W)


def _transpose_w_tc(Wt):
    BLKV = 2048

    def tr(wt_ref, o_ref):
        o_ref[:, :] = wt_ref[:, :].T

    return pl.pallas_call(
        tr,
        grid=((VOCAB + BLKV - 1) // BLKV,),
        in_specs=[pl.BlockSpec((FACT, BLKV), lambda i: (0, i))],
        out_specs=pl.BlockSpec((BLKV, FACT), lambda i: (i, 0)),
        out_shape=jax.ShapeDtypeStruct((VOCAB, FACT), jnp.float32),
    )(Wt)


def _project_tc(g, WeT):
    BLK = 2048

    def mm(g_ref, wt_ref, o_ref):
        o_ref[:, :] = jnp.dot(
            g_ref[:, :], wt_ref[:, :], preferred_element_type=jnp.float32
        )

    return pl.pallas_call(
        mm,
        grid=(N_TOKENS // BLK,),
        in_specs=[
            pl.BlockSpec((BLK, FACT), lambda i: (i, 0)),
            pl.BlockSpec((FACT, EMB), lambda i: (0, 0)),
        ],
        out_specs=pl.BlockSpec((BLK, EMB), lambda i: (i, 0)),
        out_shape=jax.ShapeDtypeStruct((N_TOKENS, EMB), jnp.float32),
    )(g, WeT)


def kernel(x, W, We):
    B, L = x.shape
    xT2d = x.T.reshape(NW * NCH, CH)
    W_rm = _transpose_w_tc(W.T)
    g = _gather_sc(xT2d, W_rm)
    out = _project_tc(g, We.T)
    return out.reshape(L, B, EMB).transpose(1, 0, 2)


# speedup vs baseline: 32.1390x; 2.4158x over previous
import functools

import jax
import jax.numpy as jnp
from jax import lax
from jax.experimental import pallas as pl
from jax.experimental.pallas import tpu as pltpu
from jax.experimental.pallas import tpu_sc as plsc

VOCAB = 1000000
EMB = 128
FACT = 32
N_TOKENS = 16384 * 50

NC = 2
NS = 16
NW = NC * NS
BPW = N_TOKENS // NW
CH = 128
NCH = BPW // CH
K_INFLIGHT = 4
SLAB = CH * K_INFLIGHT


def _project_table_tc(Wt, WeT):
    BLKV = 4096

    def mm(wt_ref, we_ref, o_ref):
        o_ref[:, :] = lax.dot_general(
            wt_ref[:, :], we_ref[:, :],
            dimension_numbers=(((0,), (0,)), ((), ())),
            preferred_element_type=jnp.float32,
        )

    return pl.pallas_call(
        mm,
        grid=((VOCAB + BLKV - 1) // BLKV,),
        in_specs=[
            pl.BlockSpec((FACT, BLKV), lambda i: (0, i)),
            pl.BlockSpec((FACT, EMB), lambda i: (0, 0)),
        ],
        out_specs=pl.BlockSpec((BLKV, EMB), lambda i: (i, 0)),
        out_shape=jax.ShapeDtypeStruct((VOCAB, EMB), jnp.float32),
    )(Wt, WeT)


def _lookup_sc(x2d, P):
    mesh = plsc.VectorSubcoreMesh(core_axis_name="c", subcore_axis_name="s")

    @functools.partial(
        pl.kernel,
        mesh=mesh,
        compiler_params=pltpu.CompilerParams(use_tc_tiling_on_sc=False),
        out_type=jax.ShapeDtypeStruct((N_TOKENS, EMB), jnp.float32),
        scratch_types=[
            pltpu.VMEM((NCH, CH), jnp.int32),
            pltpu.VMEM((SLAB, EMB), jnp.float32),
            pltpu.SemaphoreType.DMA,
        ],
    )
    def k(x_hbm, p_hbm, out_hbm, idx_v, rows_v, sem):
        wid = lax.axis_index("s") * NC + lax.axis_index("c")
        row0 = wid * BPW
        pltpu.sync_copy(x_hbm.at[pl.ds(wid * NCH, NCH), :], idx_v)
        for s in range(NCH // K_INFLIGHT):
            copies = []
            for j in range(K_INFLIGHT):
                c = s * K_INFLIGHT + j
                copies.append(
                    pltpu.async_copy(
                        p_hbm.at[idx_v.at[c]],
                        rows_v.at[pl.ds(j * CH, CH)],
                        sem,
                    )
                )
            for cp in copies:
                cp.wait()
            pltpu.sync_copy(
                rows_v, out_hbm.at[pl.ds(row0 + s * SLAB, SLAB), :]
            )

    return k(x2d, P)


def kernel(x, W, We):
    B, L = x.shape
    xT2d = x.T.reshape(NW * NCH, CH)
    P = _project_table_tc(W.T, We.T)
    out = _lookup_sc(xT2d, P)
    return out.reshape(L, B, EMB).transpose(1, 0, 2)
